# Initial kernel scaffold; baseline (speedup 1.0000x reference)
import functools

import jax
import jax.numpy as jnp
from jax import lax
from jax.experimental import pallas as pl
from jax.experimental.pallas import tpu as pltpu
from jax.experimental.pallas import tpu_sc as plsc

VOCAB = 1000000
EMBED_DIM = 32

NC = 2
NS = 16
NW = NC * NS

B_TOTAL = 4096 * 200
B_PER_W = B_TOTAL // NW
CHUNK = 1024
N_CHUNKS = B_PER_W // CHUNK


def _make_kernel():
  mesh = plsc.VectorSubcoreMesh(
      core_axis_name="c", subcore_axis_name="s", num_cores=NC,
      num_subcores=NS)

  @functools.partial(
      pl.kernel,
      out_type=jax.ShapeDtypeStruct((B_TOTAL, EMBED_DIM), jnp.float32),
      mesh=mesh,
      scratch_types=[
          pltpu.VMEM((CHUNK,), jnp.int32),
          pltpu.VMEM((CHUNK, EMBED_DIM), jnp.float32),
          pltpu.SemaphoreType.DMA,
      ],
  )
  def gather_kernel(idx_hbm, table_hbm, out_hbm, idx_v, rows_v, sem):
    wid = lax.axis_index("s") * NC + lax.axis_index("c")
    base = wid * B_PER_W

    @pl.loop(0, N_CHUNKS)
    def _chunk(c):
      off = base + c * CHUNK
      pltpu.sync_copy(idx_hbm.at[pl.ds(off, CHUNK)], idx_v)
      pltpu.async_copy(table_hbm.at[idx_v], rows_v, sem).wait()
      pltpu.sync_copy(rows_v, out_hbm.at[pl.ds(off, CHUNK)])

  return gather_kernel


_gather = _make_kernel()


@jax.jit
def kernel(token_indices, embedding_table):
  B, T = token_indices.shape
  flat_idx = token_indices.reshape((B * T,)).astype(jnp.int32)
  out = _gather(flat_idx, embedding_table)
  return out.reshape((B, T, EMBED_DIM))


# speedup vs baseline: 1.4595x; 1.4595x over previous
import functools

import jax
import jax.numpy as jnp
from jax import lax
from jax.experimental import pallas as pl
from jax.experimental.pallas import tpu as pltpu
from jax.experimental.pallas import tpu_sc as plsc

VOCAB = 1000000
EMBED_DIM = 32

NC = 2
NS = 16
NW = NC * NS

B_TOTAL = 4096 * 200
B_PER_W = B_TOTAL // NW
CHUNK = 1024
N_CHUNKS = B_PER_W // CHUNK


def _make_kernel():
  mesh = plsc.VectorSubcoreMesh(
      core_axis_name="c", subcore_axis_name="s", num_cores=NC,
      num_subcores=NS)

  @functools.partial(
      pl.kernel,
      out_type=jax.ShapeDtypeStruct((B_TOTAL, EMBED_DIM), jnp.float32),
      mesh=mesh,
      scratch_types=[
          pltpu.VMEM((CHUNK,), jnp.int32),
          pltpu.VMEM((CHUNK, EMBED_DIM), jnp.float32),
          pltpu.SemaphoreType.DMA,
      ],
      compiler_params=pltpu.CompilerParams(use_tc_tiling_on_sc=False),
  )
  def gather_kernel(idx_hbm, table_hbm, out_hbm, idx_v, rows_v, sem):
    wid = lax.axis_index("s") * NC + lax.axis_index("c")
    base = wid * B_PER_W

    @pl.loop(0, N_CHUNKS)
    def _chunk(c):
      off = base + c * CHUNK
      pltpu.sync_copy(idx_hbm.at[pl.ds(off, CHUNK)], idx_v)
      pltpu.async_copy(table_hbm.at[idx_v], rows_v, sem).wait()
      pltpu.sync_copy(rows_v, out_hbm.at[pl.ds(off, CHUNK)])

  return gather_kernel


_gather = _make_kernel()


@jax.jit
def kernel(token_indices, embedding_table):
  B, T = token_indices.shape
  flat_idx = token_indices.reshape((B * T,)).astype(jnp.int32)
  out = _gather(flat_idx, embedding_table)
  return out.reshape((B, T, EMBED_DIM))


# speedup vs baseline: 1.5013x; 1.0287x over previous
import functools

import jax
import jax.numpy as jnp
from jax import lax
from jax.experimental import pallas as pl
from jax.experimental.pallas import tpu as pltpu
from jax.experimental.pallas import tpu_sc as plsc

VOCAB = 1000000
EMBED_DIM = 32

NC = 2
NS = 16
NW = NC * NS

B_TOTAL = 4096 * 200
B_PER_W = B_TOTAL // NW
CHUNK = 1280
N_CHUNKS = B_PER_W // CHUNK


def _make_kernel():
  mesh = plsc.VectorSubcoreMesh(
      core_axis_name="c", subcore_axis_name="s", num_cores=NC,
      num_subcores=NS)

  @functools.partial(
      pl.kernel,
      out_type=jax.ShapeDtypeStruct((B_TOTAL, EMBED_DIM), jnp.float32),
      mesh=mesh,
      scratch_types=[
          pltpu.VMEM((B_PER_W,), jnp.int32),
          pltpu.VMEM((2, CHUNK, EMBED_DIM), jnp.float32),
          pltpu.SemaphoreType.DMA((2,)),
          pltpu.SemaphoreType.DMA((2,)),
      ],
      compiler_params=pltpu.CompilerParams(use_tc_tiling_on_sc=False),
  )
  def gather_kernel(idx_hbm, table_hbm, out_hbm, idx_all, rows_v, sem_g,
                    sem_o):
    wid = lax.axis_index("s") * NC + lax.axis_index("c")
    base = wid * B_PER_W
    pltpu.sync_copy(idx_hbm.at[pl.ds(base, B_PER_W)], idx_all)

    def gather(c, b):
      return pltpu.make_async_copy(
          table_hbm.at[idx_all.at[pl.ds(c * CHUNK, CHUNK)]],
          rows_v.at[b], sem_g.at[b])

    def writeback(c, b):
      return pltpu.make_async_copy(
          rows_v.at[b], out_hbm.at[pl.ds(base + c * CHUNK, CHUNK)],
          sem_o.at[b])

    gather(0, 0).start()
    gather(0, 0).wait()
    writeback(0, 0).start()
    gather(1, 1).start()

    @pl.loop(1, N_CHUNKS - 1, step=2)
    def _pair(g):
      for i in range(2):
        c = g + i
        b = (1 + i) % 2
        writeback(c - 1, (b + 1) % 2).wait()
        gather(c + 1, (b + 1) % 2).start()
        gather(c, b).wait()
        writeback(c, b).start()

    writeback(N_CHUNKS - 2, 0).wait()
    gather(N_CHUNKS - 1, 1).wait()
    writeback(N_CHUNKS - 1, 1).start()
    writeback(N_CHUNKS - 1, 1).wait()

  return gather_kernel


_gather = _make_kernel()


@jax.jit
def kernel(token_indices, embedding_table):
  B, T = token_indices.shape
  flat_idx = token_indices.reshape((B * T,)).astype(jnp.int32)
  out = _gather(flat_idx, embedding_table)
  return out.reshape((B, T, EMBED_DIM))


# speedup vs baseline: 1.5044x; 1.0021x over previous
import functools

import jax
import jax.numpy as jnp
from jax import lax
from jax.experimental import pallas as pl
from jax.experimental.pallas import tpu as pltpu
from jax.experimental.pallas import tpu_sc as plsc

VOCAB = 1000000
EMBED_DIM = 32

NC = 2
NS = 16
NW = NC * NS

B_TOTAL = 4096 * 200
B_PER_W = B_TOTAL // NW
CHUNK = 640
N_CHUNKS = B_PER_W // CHUNK
NBUF = 4


def _make_kernel():
  mesh = plsc.VectorSubcoreMesh(
      core_axis_name="c", subcore_axis_name="s", num_cores=NC,
      num_subcores=NS)

  @functools.partial(
      pl.kernel,
      out_type=jax.ShapeDtypeStruct((B_TOTAL, EMBED_DIM), jnp.float32),
      mesh=mesh,
      scratch_types=[
          pltpu.VMEM((B_PER_W,), jnp.int32),
          pltpu.VMEM((NBUF, CHUNK, EMBED_DIM), jnp.float32),
          pltpu.SemaphoreType.DMA((NBUF,)),
          pltpu.SemaphoreType.DMA((NBUF,)),
      ],
      compiler_params=pltpu.CompilerParams(use_tc_tiling_on_sc=False),
  )
  def gather_kernel(idx_hbm, table_hbm, out_hbm, idx_all, rows_v, sem_g,
                    sem_o):
    wid = lax.axis_index("s") * NC + lax.axis_index("c")
    base = wid * B_PER_W
    pltpu.sync_copy(idx_hbm.at[pl.ds(base, B_PER_W)], idx_all)

    def gather(c, b):
      return pltpu.make_async_copy(
          table_hbm.at[idx_all.at[pl.ds(c * CHUNK, CHUNK)]],
          rows_v.at[b], sem_g.at[b])

    def writeback(c, b):
      return pltpu.make_async_copy(
          rows_v.at[b], out_hbm.at[pl.ds(base + c * CHUNK, CHUNK)],
          sem_o.at[b])

    for c in range(NBUF - 1):
      gather(c, c).start()
    gather(NBUF - 1, NBUF - 1).start()
    gather(0, 0).wait()
    writeback(0, 0).start()

    @pl.loop(1, N_CHUNKS - NBUF + 1, step=NBUF)
    def _grp(g):
      for i in range(NBUF):
        c = g + i
        b = (1 + i) % NBUF
        writeback(c - 1, (b - 1) % NBUF).wait()
        gather(c + NBUF - 1, (b - 1) % NBUF).start()
        gather(c, b).wait()
        writeback(c, b).start()

    for c in range(N_CHUNKS - NBUF + 1, N_CHUNKS):
      b = c % NBUF
      writeback(c - 1, (b - 1) % NBUF).wait()
      gather(c, b).wait()
      writeback(c, b).start()
    writeback(N_CHUNKS - 1, (N_CHUNKS - 1) % NBUF).wait()

  return gather_kernel


_gather = _make_kernel()


@jax.jit
def kernel(token_indices, embedding_table):
  B, T = token_indices.shape
  flat_idx = token_indices.reshape((B * T,)).astype(jnp.int32)
  out = _gather(flat_idx, embedding_table)
  return out.reshape((B, T, EMBED_DIM))


# speedup vs baseline: 1.5052x; 1.0006x over previous
import functools

import jax
import jax.numpy as jnp
from jax import lax
from jax.experimental import pallas as pl
from jax.experimental.pallas import tpu as pltpu
from jax.experimental.pallas import tpu_sc as plsc

VOCAB = 1000000
EMBED_DIM = 32

NC = 2
NS = 16
NW = NC * NS

B = 4096
T = 200
CHUNK = 512
BLK = B // CHUNK
N_CHUNKS = (T * BLK) // NW
PER_W = N_CHUNKS * CHUNK
IDXROWS = PER_W // B + 1
NBUF = 5


def _make_kernel():
  mesh = plsc.VectorSubcoreMesh(
      core_axis_name="c", subcore_axis_name="s", num_cores=NC,
      num_subcores=NS)

  @functools.partial(
      pl.kernel,
      out_type=jax.ShapeDtypeStruct((B, T, EMBED_DIM), jnp.float32),
      mesh=mesh,
      scratch_types=[
          pltpu.VMEM((IDXROWS, B), jnp.int32),
          pltpu.VMEM((NBUF, CHUNK, EMBED_DIM), jnp.float32),
          pltpu.SemaphoreType.DMA((NBUF,)),
          pltpu.SemaphoreType.DMA((NBUF,)),
      ],
      compiler_params=pltpu.CompilerParams(use_tc_tiling_on_sc=False),
  )
  def gather_kernel(idx_hbm, table_hbm, out_hbm, idx_all, rows_v, sem_g,
                    sem_o):
    wid = lax.axis_index("s") * NC + lax.axis_index("c")
    t0 = (wid * PER_W) // B
    off0 = wid * PER_W - t0 * B
    pltpu.sync_copy(idx_hbm.at[pl.ds(t0, IDXROWS)], idx_all)

    def gather(c, b):
      p = off0 + c * CHUNK
      return pltpu.make_async_copy(
          table_hbm.at[idx_all.at[p // B, pl.ds(p % B, CHUNK)]],
          rows_v.at[b], sem_g.at[b])

    def writeback(c, b):
      g = wid * N_CHUNKS + c
      return pltpu.make_async_copy(
          rows_v.at[b],
          out_hbm.at[pl.ds((g % BLK) * CHUNK, CHUNK), g // BLK],
          sem_o.at[b])

    for c in range(NBUF):
      gather(c, c).start()
    gather(0, 0).wait()
    writeback(0, 0).start()

    @pl.loop(1, N_CHUNKS - NBUF + 1, step=NBUF)
    def _grp(g):
      for i in range(NBUF):
        c = g + i
        b = (1 + i) % NBUF
        bp = (b - 1) % NBUF
        writeback(c - 1, bp).wait()
        gather(c + NBUF - 1, bp).start()
        gather(c, b).wait()
        writeback(c, b).start()

    for c in range(N_CHUNKS - NBUF + 1, N_CHUNKS):
      b = c % NBUF
      writeback(c - 1, (b - 1) % NBUF).wait()
      gather(c, b).wait()
      writeback(c, b).start()
    writeback(N_CHUNKS - 1, (N_CHUNKS - 1) % NBUF).wait()

  return gather_kernel


_gather = _make_kernel()


@jax.jit
def kernel(token_indices, embedding_table):
  idx_t = token_indices.T.astype(jnp.int32)
  return _gather(idx_t, embedding_table)
